# baseline (speedup 1.0000x reference)
import functools

import jax
import jax.numpy as jnp
from jax import lax
from jax.experimental import pallas as pl
from jax.experimental.pallas import tpu as pltpu
from jax.experimental.pallas import tpu_sc as plsc

K = 300
KPAD = 304
NBINS = 1 << 14
HALF = NBINS // 2
BIN_SHIFT = 18
NCAND = 2048
L = 16
CH = 4
U = 4
INT_MIN = -(2 ** 31)
INT_MAX = 2 ** 31 - 1
THETA_OPT = 0x40266666


def _sortable(b):
    return b ^ lax.shift_right_logical(b >> 31, 1)


def _make_sc_call(B, Q, C):
    N = Q * C
    NP = ((N + 127) // 128) * 128
    NV = NP // L
    NVU = NV // U
    QB = ((Q * 4 + 127) // 128) * 128
    inv_c = jnp.float32(1.0 / C)

    mesh = plsc.VectorSubcoreMesh(core_axis_name="c", subcore_axis_name="s")

    @functools.partial(
        pl.kernel,
        out_type=[
            jax.ShapeDtypeStruct((B * 384,), jnp.float32),
            jax.ShapeDtypeStruct((B * 384,), jnp.int32),
            jax.ShapeDtypeStruct((B * 1536,), jnp.float32),
        ],
        mesh=mesh,
        compiler_params=pltpu.CompilerParams(
            needs_layout_passes=False, use_tc_tiling_on_sc=True),
        scratch_types=[
            pltpu.VMEM((NP,), jnp.int32),
            pltpu.VMEM((NBINS,), jnp.int32),
            pltpu.VMEM((NCAND + L,), jnp.int32),
            pltpu.VMEM((NCAND + L,), jnp.int32),
            pltpu.VMEM((NCAND,), jnp.int32),
            pltpu.VMEM((QB,), jnp.float32),
            pltpu.VMEM((128,), jnp.float32),
            pltpu.VMEM((384,), jnp.float32),
            pltpu.VMEM((384,), jnp.int32),
            pltpu.VMEM((1536,), jnp.float32),
        ],
    )
    def sc_call(keys_hbm, boxes_hbm, scale_hbm,
                out_s_hbm, out_l_hbm, out_b_hbm,
                keys_v, hist_v, cand_s_v, cand_i_v, rank_v,
                boxes_v, scale_v, s_st, l_st, b_st):
        img = lax.axis_index("s") * 2 + lax.axis_index("c")

        pltpu.sync_copy(keys_hbm.at[pl.ds(img * NP, NP)], keys_v)
        pltpu.sync_copy(boxes_hbm.at[pl.ds(img * QB, QB)], boxes_v)
        pltpu.sync_copy(scale_hbm.at[pl.ds(img * 128, 128)], scale_v)

        lanes = lax.iota(jnp.int32, L)
        ones = jnp.ones((L,), jnp.int32)
        zeros = jnp.zeros((L,), jnp.int32)

        def zero_hist(i, _):
            for u in range(4):
                hist_v[pl.ds((i * 4 + u) * L, L)] = zeros
            return 0
        lax.fori_loop(0, NBINS // L // 4, zero_hist, 0)

        def p1_step(vi, wptr_v):
            s = keys_v[pl.ds(vi * L, L)]
            bins = (s >> BIN_SHIFT) + HALF
            plsc.addupdate_scatter(hist_v, [bins], ones)
            m = s >= THETA_OPT
            wp = jnp.minimum(wptr_v[0], NCAND - L)
            plsc.store_compressed(cand_i_v.at[pl.ds(wp, L)],
                                  vi * L + lanes, mask=m)
            return wptr_v + plsc.all_reduce_population_count(m)

        def pass1(i, wptr_v):
            for u in range(U):
                wptr_v = p1_step(i * U + u, wptr_v)
            return wptr_v
        wptr_v = lax.fori_loop(0, NVU, pass1, zeros)
        for u in range(NVU * U, NV):
            wptr_v = p1_step(u, wptr_v)
        n_opt = wptr_v[0]

        def scan_cond(c):
            prev, cum, vr = c
            return jnp.logical_and(cum < K, vr >= 0)

        def scan_chunk(c):
            prev, cum, vr = c
            acc = zeros
            for k in range(CH):
                acc = acc + hist_v[pl.ds((vr + k) * L, L)]
            return cum, cum + jnp.sum(acc), vr - CH
        prev, _, vr_exit = lax.while_loop(
            scan_cond, scan_chunk,
            (jnp.int32(0), jnp.int32(0), jnp.int32(NBINS // L - CH)))
        cbase = vr_exit + CH

        def fine_scan(j, carry):
            cum, bstar = carry
            vr = cbase + CH - 1 - j
            v = hist_v[pl.ds(vr * L, L)]
            sfx = jnp.cumsum(lax.rev(v, (0,)))
            tot = jnp.sum(v)
            p = jnp.sum((cum + sfx < K).astype(jnp.int32))
            newcum = cum + tot
            crossed = jnp.logical_and(cum < K, newcum >= K)
            bstar = jnp.where(crossed, vr * L + (L - 1) - p, bstar)
            return newcum, bstar
        _, bstar = lax.fori_loop(0, CH, fine_scan, (prev, jnp.int32(0)))
        theta = lax.shift_left(bstar - HALF, BIN_SHIFT)

        good = jnp.logical_and(theta >= THETA_OPT, n_opt <= NCAND - L)

        def mini_compact(_):
            nv_opt = (n_opt + L - 1) // L
            def body(i, wptr_v):
                g = i * L + lanes
                valid = g < n_opt
                ci = cand_i_v[pl.ds(i * L, L)]
                ci = jnp.where(valid, ci, 0)
                s = plsc.load_gather(keys_v, [ci])
                m = jnp.logical_and(s >= theta, valid)
                wp = wptr_v[0]
                plsc.store_compressed(cand_s_v.at[pl.ds(wp, L)], s, mask=m)
                plsc.store_compressed(cand_i_v.at[pl.ds(wp, L)], ci, mask=m)
                return wptr_v + plsc.all_reduce_population_count(m)
            return lax.fori_loop(0, nv_opt, body, zeros)[0]

        def full_compact(_):
            def body(i, wptr_v):
                s = keys_v[pl.ds(i * L, L)]
                m = s >= theta
                wp = jnp.minimum(wptr_v[0], NCAND - L)
                plsc.store_compressed(cand_s_v.at[pl.ds(wp, L)], s, mask=m)
                plsc.store_compressed(cand_i_v.at[pl.ds(wp, L)],
                                      i * L + lanes, mask=m)
                return wptr_v + plsc.all_reduce_population_count(m)
            return lax.fori_loop(0, NV, body, zeros)[0]

        n = lax.cond(good, mini_compact, full_compact, 0)
        n = jnp.minimum(n, NCAND)
        cand_s_v[pl.ds(n, L)] = jnp.full((L,), INT_MIN, jnp.int32)
        cand_i_v[pl.ds(n, L)] = jnp.full((L,), INT_MAX, jnp.int32)
        nv = (n + L - 1) // L

        def rank_outer(iv, _):
            sv = cand_s_v[pl.ds(iv * L, L)]
            ivv = cand_i_v[pl.ds(iv * L, L)]
            def rank_inner(jv, racc):
                sjv = cand_s_v[pl.ds(jv * L, L)]
                ijv = cand_i_v[pl.ds(jv * L, L)]
                for k in range(L):
                    sj = sjv[k]
                    ij = ijv[k]
                    beats = jnp.logical_or(
                        sj > sv, jnp.logical_and(sj == sv, ij < ivv))
                    racc = racc + beats.astype(jnp.int32)
                return racc
            rank_v[pl.ds(iv * L, L)] = lax.fori_loop(0, nv, rank_inner, zeros)
            return 0
        lax.fori_loop(0, nv, rank_outer, 0)

        scale_vec = scale_v[pl.ds(0, L)]
        img_w = scale_vec[0]
        img_h = scale_vec[1]

        def emit(iv, _):
            base = iv * L
            r = rank_v[pl.ds(base, L)]
            s = cand_s_v[pl.ds(base, L)]
            ci = cand_i_v[pl.ds(base, L)]
            m = r < K
            rr = jnp.where(m, r, 0)
            x = lax.bitcast_convert_type(_sortable(s), jnp.float32)
            score = 1.0 / (1.0 + jnp.exp(-x))
            q = ((ci.astype(jnp.float32) + 0.5) * inv_c).astype(jnp.int32)
            q = jnp.where(m, q, 0)
            label = ci - q * C
            qb = q * 4
            cx = plsc.load_gather(boxes_v, [qb])
            cy = plsc.load_gather(boxes_v, [qb + 1])
            w = jnp.maximum(plsc.load_gather(boxes_v, [qb + 2]), 0.0)
            h = jnp.maximum(plsc.load_gather(boxes_v, [qb + 3]), 0.0)
            plsc.store_scatter(s_st, [rr], score, mask=m)
            plsc.store_scatter(l_st, [rr], label, mask=m)
            rb = rr * 4
            plsc.store_scatter(b_st, [rb], (cx - 0.5 * w) * img_w, mask=m)
            plsc.store_scatter(b_st, [rb + 1], (cy - 0.5 * h) * img_h, mask=m)
            plsc.store_scatter(b_st, [rb + 2], (cx + 0.5 * w) * img_w, mask=m)
            plsc.store_scatter(b_st, [rb + 3], (cy + 0.5 * h) * img_h, mask=m)
            return 0
        lax.fori_loop(0, nv, emit, 0)

        pltpu.sync_copy(s_st, out_s_hbm.at[pl.ds(img * 384, 384)])
        pltpu.sync_copy(l_st, out_l_hbm.at[pl.ds(img * 384, 384)])
        pltpu.sync_copy(b_st, out_b_hbm.at[pl.ds(img * 1536, 1536)])

    return sc_call


def kernel(pred_logits, pred_boxes, target_sizes):
    B, Q, C = pred_logits.shape
    N = Q * C
    NP = ((N + 127) // 128) * 128
    QB = ((Q * 4 + 127) // 128) * 128
    keys = _sortable(
        lax.bitcast_convert_type(pred_logits, jnp.int32)).reshape(B, N)
    keys1d = jnp.pad(keys, ((0, 0), (0, NP - N)),
                     constant_values=INT_MIN).reshape(B * NP)
    boxes1d = jnp.pad(pred_boxes.reshape(B, Q * 4),
                      ((0, 0), (0, QB - Q * 4))).reshape(B * QB)
    ts = target_sizes.astype(jnp.float32)
    scale1d = jnp.pad(jnp.stack([ts[:, 1], ts[:, 0]], axis=1),
                      ((0, 0), (0, 126))).reshape(B * 128)
    s_pad, l_pad, b_pad = _make_sc_call(B, Q, C)(keys1d, boxes1d, scale1d)
    scores = s_pad.reshape(B, 384)[:, :K]
    labels = l_pad.reshape(B, 384)[:, :K]
    boxes = b_pad.reshape(B, 384, 4)[:, :K, :]
    return scores, labels, boxes


# speedup vs baseline: 1.2692x; 1.2692x over previous
import functools

import jax
import jax.numpy as jnp
from jax import lax
from jax.experimental import pallas as pl
from jax.experimental.pallas import tpu as pltpu
from jax.experimental.pallas import tpu_sc as plsc

K = 300
NBINS = 1 << 14
HALF = NBINS // 2
BIN_SHIFT = 18
NCAND = 2048
L = 16
CH = 4
RCH = 200
NCHUNK = 5
KSP = 384
INT_MIN = -(2 ** 31)
INT_MAX = 2 ** 31 - 1
THETA_OPT = 0x40266666


def _sortable(b):
    return b ^ lax.shift_right_logical(b >> 31, 1)


_C0S = (0, 16, 32, 48, 64, 75)


def _make_sc_call(B, Q, C):
    QB = Q * 4
    inv_c = jnp.float32(1.0 / C)

    mesh = plsc.VectorSubcoreMesh(core_axis_name="c", subcore_axis_name="s")

    @functools.partial(
        pl.kernel,
        out_type=[
            jax.ShapeDtypeStruct((B * KSP,), jnp.float32),
            jax.ShapeDtypeStruct((B * KSP,), jnp.int32),
            jax.ShapeDtypeStruct((B * KSP * 4,), jnp.float32),
        ],
        mesh=mesh,
        compiler_params=pltpu.CompilerParams(
            needs_layout_passes=False, use_tc_tiling_on_sc=True),
        scratch_types=[
            pltpu.VMEM((RCH, C), jnp.float32),
            pltpu.VMEM((RCH, C), jnp.float32),
            pltpu.VMEM((NBINS,), jnp.int32),
            pltpu.VMEM((NCAND + L,), jnp.int32),
            pltpu.VMEM((NCAND + L,), jnp.int32),
            pltpu.VMEM((NCAND,), jnp.int32),
            pltpu.VMEM((QB,), jnp.float32),
            pltpu.VMEM((KSP,), jnp.float32),
            pltpu.VMEM((KSP,), jnp.int32),
            pltpu.VMEM((KSP * 4,), jnp.float32),
            pltpu.SemaphoreType.DMA,
            pltpu.SemaphoreType.DMA,
        ],
    )
    def sc_call(logits_hbm, boxes_hbm,
                out_s_hbm, out_l_hbm, out_b_hbm,
                buf_a, buf_b, hist_v, cand_s_v, cand_i_v, rank_v,
                boxes_v, s_st, l_st, b_st, sem_a, sem_b):
        img = lax.axis_index("s") * 2 + lax.axis_index("c")

        bufs = (buf_a, buf_b)
        sems = (sem_a, sem_b)
        copies = [None, None]

        def start_chunk(c):
            copies[c % 2] = pltpu.async_copy(
                logits_hbm.at[img, pl.ds(c * RCH, RCH)], bufs[c % 2],
                sems[c % 2])

        start_chunk(0)
        pltpu.sync_copy(boxes_hbm.at[pl.ds(img * QB, QB)], boxes_v)

        lanes = lax.iota(jnp.int32, L)
        ones = jnp.ones((L,), jnp.int32)
        zeros = jnp.zeros((L,), jnp.int32)
        cvecs = [c0 + lanes for c0 in _C0S]
        last_ok = lanes >= 5

        def zero_hist(i, _):
            for u in range(4):
                hist_v[pl.ds((i * 4 + u) * L, L)] = zeros
            return 0
        lax.fori_loop(0, NBINS // L // 4, zero_hist, 0)

        def p1_row(buf, r, base91, wptr_v):
            for si, c0 in enumerate(_C0S):
                x = buf[r, pl.ds(c0, L)]
                b = lax.bitcast_convert_type(x, jnp.int32)
                s = _sortable(b)
                bins = (s >> BIN_SHIFT) + HALF
                m = s >= THETA_OPT
                if si == 5:
                    plsc.addupdate_scatter(hist_v, [bins], ones, mask=last_ok)
                    m = jnp.logical_and(m, last_ok)
                else:
                    plsc.addupdate_scatter(hist_v, [bins], ones)
                wp = jnp.minimum(wptr_v[0], NCAND - L)
                plsc.store_compressed(cand_s_v.at[pl.ds(wp, L)], s, mask=m)
                plsc.store_compressed(cand_i_v.at[pl.ds(wp, L)],
                                      base91 + cvecs[si], mask=m)
                wptr_v = wptr_v + plsc.all_reduce_population_count(m)
            return wptr_v

        wptr_v = zeros
        for c in range(NCHUNK):
            if c + 1 < NCHUNK:
                start_chunk(c + 1)
            copies[c % 2].wait()
            buf = bufs[c % 2]

            def p1_rows(r, wv, _buf=buf, _c=c):
                return p1_row(_buf, r, (_c * RCH + r) * C, wv)
            wptr_v = lax.fori_loop(0, RCH, p1_rows, wptr_v)
        n_opt = wptr_v[0]

        def scan_cond(carry):
            prev, cum, vr = carry
            return jnp.logical_and(cum < K, vr >= 0)

        def scan_chunk(carry):
            prev, cum, vr = carry
            acc = zeros
            for k in range(CH):
                acc = acc + hist_v[pl.ds((vr + k) * L, L)]
            return cum, cum + jnp.sum(acc), vr - CH
        prev, _, vr_exit = lax.while_loop(
            scan_cond, scan_chunk,
            (jnp.int32(0), jnp.int32(0), jnp.int32(NBINS // L - CH)))
        cbase = vr_exit + CH

        def fine_scan(j, carry):
            cum, bstar = carry
            vr = cbase + CH - 1 - j
            v = hist_v[pl.ds(vr * L, L)]
            sfx = jnp.cumsum(lax.rev(v, (0,)))
            tot = jnp.sum(v)
            p = jnp.sum((cum + sfx < K).astype(jnp.int32))
            newcum = cum + tot
            crossed = jnp.logical_and(cum < K, newcum >= K)
            bstar = jnp.where(crossed, vr * L + (L - 1) - p, bstar)
            return newcum, bstar
        _, bstar = lax.fori_loop(0, CH, fine_scan, (prev, jnp.int32(0)))
        theta = lax.shift_left(bstar - HALF, BIN_SHIFT)

        good = jnp.logical_and(theta >= THETA_OPT, n_opt <= NCAND - L)

        def mini_compact(_):
            nv_opt = (n_opt + L - 1) // L
            def body(i, wv):
                g = i * L + lanes
                s = cand_s_v[pl.ds(i * L, L)]
                ci = cand_i_v[pl.ds(i * L, L)]
                m = jnp.logical_and(s >= theta, g < n_opt)
                wp = wv[0]
                plsc.store_compressed(cand_s_v.at[pl.ds(wp, L)], s, mask=m)
                plsc.store_compressed(cand_i_v.at[pl.ds(wp, L)], ci, mask=m)
                return wv + plsc.all_reduce_population_count(m)
            return lax.fori_loop(0, nv_opt, body, zeros)[0]

        def full_compact(_):
            wv = zeros
            for c in range(NCHUNK):
                pltpu.sync_copy(
                    logits_hbm.at[img, pl.ds(c * RCH, RCH)], buf_a)

                def fc_row(r, w, _c=c):
                    base91 = (_c * RCH + r) * C
                    for si, c0 in enumerate(_C0S):
                        x = buf_a[r, pl.ds(c0, L)]
                        s = _sortable(lax.bitcast_convert_type(x, jnp.int32))
                        m = s >= theta
                        if si == 5:
                            m = jnp.logical_and(m, last_ok)
                        wp = jnp.minimum(w[0], NCAND - L)
                        plsc.store_compressed(
                            cand_s_v.at[pl.ds(wp, L)], s, mask=m)
                        plsc.store_compressed(
                            cand_i_v.at[pl.ds(wp, L)], base91 + cvecs[si],
                            mask=m)
                        w = w + plsc.all_reduce_population_count(m)
                    return w
                wv = lax.fori_loop(0, RCH, fc_row, wv)
            return wv[0]

        n = lax.cond(good, mini_compact, full_compact, 0)
        n = jnp.minimum(n, NCAND)
        cand_s_v[pl.ds(n, L)] = jnp.full((L,), INT_MIN, jnp.int32)
        cand_i_v[pl.ds(n, L)] = jnp.full((L,), INT_MAX, jnp.int32)
        nv = (n + L - 1) // L

        def rank_outer(iv, _):
            sv = cand_s_v[pl.ds(iv * L, L)]
            ivv = cand_i_v[pl.ds(iv * L, L)]
            def rank_inner(jv, racc):
                sjv = cand_s_v[pl.ds(jv * L, L)]
                ijv = cand_i_v[pl.ds(jv * L, L)]
                for k in range(L):
                    sj = sjv[k]
                    ij = ijv[k]
                    beats = jnp.logical_or(
                        sj > sv, jnp.logical_and(sj == sv, ij < ivv))
                    racc = racc + beats.astype(jnp.int32)
                return racc
            rank_v[pl.ds(iv * L, L)] = lax.fori_loop(0, nv, rank_inner, zeros)
            return 0
        lax.fori_loop(0, nv, rank_outer, 0)

        def emit(iv, _):
            base = iv * L
            r = rank_v[pl.ds(base, L)]
            s = cand_s_v[pl.ds(base, L)]
            ci = cand_i_v[pl.ds(base, L)]
            m = r < K
            rr = jnp.where(m, r, 0)
            x = lax.bitcast_convert_type(_sortable(s), jnp.float32)
            score = 1.0 / (1.0 + jnp.exp(-x))
            q = ((ci.astype(jnp.float32) + 0.5) * inv_c).astype(jnp.int32)
            q = jnp.where(m, q, 0)
            label = ci - q * C
            qb = q * 4
            cx = plsc.load_gather(boxes_v, [qb])
            cy = plsc.load_gather(boxes_v, [qb + 1])
            w = jnp.maximum(plsc.load_gather(boxes_v, [qb + 2]), 0.0)
            h = jnp.maximum(plsc.load_gather(boxes_v, [qb + 3]), 0.0)
            plsc.store_scatter(s_st, [rr], score, mask=m)
            plsc.store_scatter(l_st, [rr], label, mask=m)
            rb = rr * 4
            plsc.store_scatter(b_st, [rb], cx - 0.5 * w, mask=m)
            plsc.store_scatter(b_st, [rb + 1], cy - 0.5 * h, mask=m)
            plsc.store_scatter(b_st, [rb + 2], cx + 0.5 * w, mask=m)
            plsc.store_scatter(b_st, [rb + 3], cy + 0.5 * h, mask=m)
            return 0
        lax.fori_loop(0, nv, emit, 0)

        pltpu.sync_copy(s_st, out_s_hbm.at[pl.ds(img * KSP, KSP)])
        pltpu.sync_copy(l_st, out_l_hbm.at[pl.ds(img * KSP, KSP)])
        pltpu.sync_copy(b_st, out_b_hbm.at[pl.ds(img * KSP * 4, KSP * 4)])

    return sc_call


def kernel(pred_logits, pred_boxes, target_sizes):
    B, Q, C = pred_logits.shape
    boxes1d = pred_boxes.reshape(B * Q * 4)
    s_pad, l_pad, b_pad = _make_sc_call(B, Q, C)(pred_logits, boxes1d)
    scores = s_pad.reshape(B, KSP)[:, :K]
    labels = l_pad.reshape(B, KSP)[:, :K]
    ts = target_sizes.astype(jnp.float32)
    scale_fct = jnp.stack([ts[:, 1], ts[:, 0], ts[:, 1], ts[:, 0]],
                          axis=-1)[:, None, :]
    boxes = b_pad.reshape(B, KSP, 4)[:, :K, :] * scale_fct
    return scores, labels, boxes


# speedup vs baseline: 1.6412x; 1.2930x over previous
import functools

import jax
import jax.numpy as jnp
from jax import lax
from jax.experimental import pallas as pl
from jax.experimental.pallas import tpu as pltpu
from jax.experimental.pallas import tpu_sc as plsc

K = 300
NBINS = 1 << 14
HALF = NBINS // 2
BIN_SHIFT = 18
NCAND = 2048
CAP_SI = 512
L = 16
CH = 4
RCH = 200
NCHUNK = 5
KSP = 384
INT_MIN = -(2 ** 31)
INT_MAX = 2 ** 31 - 1
THETA_OPT = 0x40266666


def _sortable(b):
    return b ^ lax.shift_right_logical(b >> 31, 1)


_C0S = (0, 16, 32, 48, 64, 75)


def _make_sc_call(B, Q, C):
    QB = Q * 4
    inv_c = jnp.float32(1.0 / C)

    mesh = plsc.VectorSubcoreMesh(core_axis_name="c", subcore_axis_name="s")

    @functools.partial(
        pl.kernel,
        out_type=[
            jax.ShapeDtypeStruct((B * KSP,), jnp.float32),
            jax.ShapeDtypeStruct((B * KSP,), jnp.int32),
            jax.ShapeDtypeStruct((B * KSP * 4,), jnp.float32),
        ],
        mesh=mesh,
        compiler_params=pltpu.CompilerParams(
            needs_layout_passes=False, use_tc_tiling_on_sc=True,
            skip_device_barrier=True),
        scratch_types=[
            pltpu.VMEM((RCH, C), jnp.float32),
            pltpu.VMEM((RCH, C), jnp.float32),
            pltpu.VMEM((NBINS,), jnp.int32),
            pltpu.VMEM((NCAND + L,), jnp.int32),
            pltpu.VMEM((NCAND + L,), jnp.int32),
        ] + [pltpu.VMEM((CAP_SI + L,), jnp.int32)
             for _ in range(6)
        ] + [pltpu.VMEM((CAP_SI + L,), jnp.int32)
             for _ in range(6)
        ] + [
            pltpu.VMEM((NCAND,), jnp.int32),
            pltpu.VMEM((QB,), jnp.float32),
            pltpu.VMEM((KSP,), jnp.float32),
            pltpu.VMEM((KSP,), jnp.int32),
            pltpu.VMEM((KSP * 4,), jnp.float32),
            pltpu.SemaphoreType.DMA,
            pltpu.SemaphoreType.DMA,
        ],
    )
    def sc_call(logits_hbm, boxes_hbm,
                out_s_hbm, out_l_hbm, out_b_hbm,
                buf_a, buf_b, hist_v, cand_s_v, cand_i_v,
                os0, os1, os2, os3, os4, os5,
                oi0, oi1, oi2, oi3, oi4, oi5, rank_v,
                boxes_v, s_st, l_st, b_st, sem_a, sem_b):
        opt_s_v = (os0, os1, os2, os3, os4, os5)
        opt_i_v = (oi0, oi1, oi2, oi3, oi4, oi5)
        img = lax.axis_index("s") * 2 + lax.axis_index("c")

        bufs = (buf_a, buf_b)
        sems = (sem_a, sem_b)
        copies = [None, None]

        def start_chunk(c):
            copies[c % 2] = pltpu.async_copy(
                logits_hbm.at[img, pl.ds(c * RCH, RCH)], bufs[c % 2],
                sems[c % 2])

        start_chunk(0)
        pltpu.sync_copy(boxes_hbm.at[pl.ds(img * QB, QB)], boxes_v)

        lanes = lax.iota(jnp.int32, L)
        ones = jnp.ones((L,), jnp.int32)
        zeros = jnp.zeros((L,), jnp.int32)
        cvecs = [c0 + lanes for c0 in _C0S]
        last_ok = lanes >= 5

        def zero_hist(i, _):
            for u in range(4):
                hist_v[pl.ds((i * 4 + u) * L, L)] = zeros
            return 0
        lax.fori_loop(0, NBINS // L // 4, zero_hist, 0)

        def p1_row(buf, r, base91, wptrs):
            xs = [buf[r, pl.ds(c0, L)] for c0 in _C0S]
            ss = [_sortable(lax.bitcast_convert_type(x, jnp.int32))
                  for x in xs]
            ms = [s >= THETA_OPT for s in ss]
            ms[5] = jnp.logical_and(ms[5], last_ok)
            new = []
            for si in range(6):
                wp = jnp.minimum(wptrs[si][0], CAP_SI - L)
                plsc.store_compressed(opt_s_v[si].at[pl.ds(wp, L)],
                                      ss[si], mask=ms[si])
                plsc.store_compressed(opt_i_v[si].at[pl.ds(wp, L)],
                                      base91 + cvecs[si], mask=ms[si])
                new.append(wptrs[si] +
                           plsc.all_reduce_population_count(ms[si]))
            return tuple(new)

        wptrs = (zeros,) * 6
        for c in range(NCHUNK):
            if c + 1 < NCHUNK:
                start_chunk(c + 1)
            copies[c % 2].wait()
            buf = bufs[c % 2]

            def p1_rows(r, wv, _buf=buf, _c=c):
                return p1_row(_buf, r, (_c * RCH + r) * C, wv)
            wptrs = lax.fori_loop(0, RCH, p1_rows, wptrs)
        n_sis = [wptrs[si][0] for si in range(6)]
        n_opt = n_sis[0] + n_sis[1] + n_sis[2] + n_sis[3] + n_sis[4] + n_sis[5]
        ok_caps = jnp.logical_and(
            jnp.logical_and(
                jnp.logical_and(n_sis[0] <= CAP_SI - L, n_sis[1] <= CAP_SI - L),
                jnp.logical_and(n_sis[2] <= CAP_SI - L, n_sis[3] <= CAP_SI - L)),
            jnp.logical_and(n_sis[4] <= CAP_SI - L, n_sis[5] <= CAP_SI - L))
        good = jnp.logical_and(n_opt >= K, ok_caps)

        def hist_from_candidates(_):
            for si in range(6):
                nv_si = (n_sis[si] + L - 1) // L
                def body(i, __, _si=si):
                    g = i * L + lanes
                    s = opt_s_v[_si][pl.ds(i * L, L)]
                    bins = (s >> BIN_SHIFT) + HALF
                    m = g < n_sis[_si]
                    bins = jnp.where(m, bins, 0)
                    plsc.addupdate_scatter(hist_v, [bins], ones, mask=m)
                    return 0
                lax.fori_loop(0, nv_si, body, 0)
            return 0

        def hist_from_stream(_):
            for c in range(NCHUNK):
                pltpu.sync_copy(
                    logits_hbm.at[img, pl.ds(c * RCH, RCH)], buf_a)
                def fh_row(r, __):
                    for si, c0 in enumerate(_C0S):
                        x = buf_a[r, pl.ds(c0, L)]
                        s = _sortable(lax.bitcast_convert_type(x, jnp.int32))
                        bins = (s >> BIN_SHIFT) + HALF
                        if si == 5:
                            bins = jnp.where(last_ok, bins, 0)
                            plsc.addupdate_scatter(hist_v, [bins], ones,
                                                   mask=last_ok)
                        else:
                            plsc.addupdate_scatter(hist_v, [bins], ones)
                    return 0
                lax.fori_loop(0, RCH, fh_row, 0)
            return 0

        lax.cond(good, hist_from_candidates, hist_from_stream, 0)

        def scan_cond(carry):
            prev, cum, vr = carry
            return jnp.logical_and(cum < K, vr >= 0)

        def scan_chunk(carry):
            prev, cum, vr = carry
            acc = zeros
            for k in range(CH):
                acc = acc + hist_v[pl.ds((vr + k) * L, L)]
            return cum, cum + jnp.sum(acc), vr - CH
        prev, _, vr_exit = lax.while_loop(
            scan_cond, scan_chunk,
            (jnp.int32(0), jnp.int32(0), jnp.int32(NBINS // L - CH)))
        cbase = vr_exit + CH

        def fine_scan(j, carry):
            cum, bstar = carry
            vr = cbase + CH - 1 - j
            v = hist_v[pl.ds(vr * L, L)]
            sfx = jnp.cumsum(lax.rev(v, (0,)))
            tot = jnp.sum(v)
            p = jnp.sum((cum + sfx < K).astype(jnp.int32))
            newcum = cum + tot
            crossed = jnp.logical_and(cum < K, newcum >= K)
            bstar = jnp.where(crossed, vr * L + (L - 1) - p, bstar)
            return newcum, bstar
        _, bstar = lax.fori_loop(0, CH, fine_scan, (prev, jnp.int32(0)))
        theta = lax.shift_left(bstar - HALF, BIN_SHIFT)

        def mini_compact(_):
            wv = zeros
            for si in range(6):
                nv_si = (n_sis[si] + L - 1) // L
                def body(i, w, _si=si):
                    g = i * L + lanes
                    s = opt_s_v[_si][pl.ds(i * L, L)]
                    ci = opt_i_v[_si][pl.ds(i * L, L)]
                    m = jnp.logical_and(s >= theta, g < n_sis[_si])
                    wp = w[0]
                    plsc.store_compressed(cand_s_v.at[pl.ds(wp, L)], s, mask=m)
                    plsc.store_compressed(cand_i_v.at[pl.ds(wp, L)], ci, mask=m)
                    return w + plsc.all_reduce_population_count(m)
                wv = lax.fori_loop(0, nv_si, body, wv)
            return wv[0]

        def full_compact(_):
            wv = zeros
            for c in range(NCHUNK):
                pltpu.sync_copy(
                    logits_hbm.at[img, pl.ds(c * RCH, RCH)], buf_a)

                def fc_row(r, w, _c=c):
                    base91 = (_c * RCH + r) * C
                    for si, c0 in enumerate(_C0S):
                        x = buf_a[r, pl.ds(c0, L)]
                        s = _sortable(lax.bitcast_convert_type(x, jnp.int32))
                        m = s >= theta
                        if si == 5:
                            m = jnp.logical_and(m, last_ok)
                        wp = jnp.minimum(w[0], NCAND - L)
                        plsc.store_compressed(
                            cand_s_v.at[pl.ds(wp, L)], s, mask=m)
                        plsc.store_compressed(
                            cand_i_v.at[pl.ds(wp, L)], base91 + cvecs[si],
                            mask=m)
                        w = w + plsc.all_reduce_population_count(m)
                    return w
                wv = lax.fori_loop(0, RCH, fc_row, wv)
            return wv[0]

        n = lax.cond(good, mini_compact, full_compact, 0)
        n = jnp.minimum(n, NCAND)
        cand_s_v[pl.ds(n, L)] = jnp.full((L,), INT_MIN, jnp.int32)
        cand_i_v[pl.ds(n, L)] = jnp.full((L,), INT_MAX, jnp.int32)
        nv = (n + L - 1) // L

        def rank_outer(iv, _):
            sv = cand_s_v[pl.ds(iv * L, L)]
            ivv = cand_i_v[pl.ds(iv * L, L)]
            def rank_inner(jv, racc):
                sjv = cand_s_v[pl.ds(jv * L, L)]
                ijv = cand_i_v[pl.ds(jv * L, L)]
                for k in range(L):
                    sj = sjv[k]
                    ij = ijv[k]
                    beats = jnp.logical_or(
                        sj > sv, jnp.logical_and(sj == sv, ij < ivv))
                    racc = racc + beats.astype(jnp.int32)
                return racc
            rank_v[pl.ds(iv * L, L)] = lax.fori_loop(0, nv, rank_inner, zeros)
            return 0
        lax.fori_loop(0, nv, rank_outer, 0)

        def emit(iv, _):
            base = iv * L
            r = rank_v[pl.ds(base, L)]
            s = cand_s_v[pl.ds(base, L)]
            ci = cand_i_v[pl.ds(base, L)]
            m = r < K
            rr = jnp.where(m, r, 0)
            x = lax.bitcast_convert_type(_sortable(s), jnp.float32)
            score = 1.0 / (1.0 + jnp.exp(-x))
            q = ((ci.astype(jnp.float32) + 0.5) * inv_c).astype(jnp.int32)
            q = jnp.where(m, q, 0)
            label = ci - q * C
            qb = q * 4
            cx = plsc.load_gather(boxes_v, [qb])
            cy = plsc.load_gather(boxes_v, [qb + 1])
            w = jnp.maximum(plsc.load_gather(boxes_v, [qb + 2]), 0.0)
            h = jnp.maximum(plsc.load_gather(boxes_v, [qb + 3]), 0.0)
            plsc.store_scatter(s_st, [rr], score, mask=m)
            plsc.store_scatter(l_st, [rr], label, mask=m)
            rb = rr * 4
            plsc.store_scatter(b_st, [rb], cx - 0.5 * w, mask=m)
            plsc.store_scatter(b_st, [rb + 1], cy - 0.5 * h, mask=m)
            plsc.store_scatter(b_st, [rb + 2], cx + 0.5 * w, mask=m)
            plsc.store_scatter(b_st, [rb + 3], cy + 0.5 * h, mask=m)
            return 0
        lax.fori_loop(0, nv, emit, 0)

        pltpu.sync_copy(s_st, out_s_hbm.at[pl.ds(img * KSP, KSP)])
        pltpu.sync_copy(l_st, out_l_hbm.at[pl.ds(img * KSP, KSP)])
        pltpu.sync_copy(b_st, out_b_hbm.at[pl.ds(img * KSP * 4, KSP * 4)])

    return sc_call


def kernel(pred_logits, pred_boxes, target_sizes):
    B, Q, C = pred_logits.shape
    boxes1d = pred_boxes.reshape(B * Q * 4)
    s_pad, l_pad, b_pad = _make_sc_call(B, Q, C)(pred_logits, boxes1d)
    scores = s_pad.reshape(B, KSP)[:, :K]
    labels = l_pad.reshape(B, KSP)[:, :K]
    ts = target_sizes.astype(jnp.float32)
    scale_fct = jnp.stack([ts[:, 1], ts[:, 0], ts[:, 1], ts[:, 0]],
                          axis=-1)[:, None, :]
    boxes = b_pad.reshape(B, KSP, 4)[:, :K, :] * scale_fct
    return scores, labels, boxes


# speedup vs baseline: 1.6824x; 1.0251x over previous
import functools

import jax
import jax.numpy as jnp
from jax import lax
from jax.experimental import pallas as pl
from jax.experimental.pallas import tpu as pltpu
from jax.experimental.pallas import tpu_sc as plsc

K = 300
NBINS = 1 << 14
HALF = NBINS // 2
BIN_SHIFT = 18
FINE_SHIFT = 7
NCAND = 2048
CAP_SI = 512
L = 16
CH = 4
RCH = 200
NCHUNK = 5
KSP = 384
INT_MIN = -(2 ** 31)
INT_MAX = 2 ** 31 - 1
THETA_OPT = 0x40266666


def _sortable(b):
    return b ^ lax.shift_right_logical(b >> 31, 1)


_C0S = (0, 16, 32, 48, 64, 75)


def _make_sc_call(B, Q, C):
    QB = Q * 4
    inv_c = jnp.float32(1.0 / C)

    mesh = plsc.VectorSubcoreMesh(core_axis_name="c", subcore_axis_name="s")

    @functools.partial(
        pl.kernel,
        out_type=[
            jax.ShapeDtypeStruct((B * KSP,), jnp.float32),
            jax.ShapeDtypeStruct((B * KSP,), jnp.int32),
            jax.ShapeDtypeStruct((B * KSP * 4,), jnp.float32),
        ],
        mesh=mesh,
        compiler_params=pltpu.CompilerParams(
            needs_layout_passes=False, use_tc_tiling_on_sc=True,
            skip_device_barrier=True),
        scratch_types=[
            pltpu.VMEM((RCH, C), jnp.float32),
            pltpu.VMEM((RCH, C), jnp.float32),
            pltpu.VMEM((NBINS,), jnp.int32),
            pltpu.VMEM((NCAND + L,), jnp.int32),
            pltpu.VMEM((NCAND + L,), jnp.int32),
        ] + [pltpu.VMEM((CAP_SI + L,), jnp.int32)
             for _ in range(6)
        ] + [pltpu.VMEM((CAP_SI + L,), jnp.int32)
             for _ in range(6)
        ] + [
            pltpu.VMEM((NCAND,), jnp.int32),
            pltpu.VMEM((QB,), jnp.float32),
            pltpu.VMEM((KSP,), jnp.float32),
            pltpu.VMEM((KSP,), jnp.int32),
            pltpu.VMEM((KSP * 4,), jnp.float32),
            pltpu.SemaphoreType.DMA,
            pltpu.SemaphoreType.DMA,
        ],
    )
    def sc_call(logits_hbm, boxes_hbm,
                out_s_hbm, out_l_hbm, out_b_hbm,
                buf_a, buf_b, hist_v, cand_s_v, cand_i_v,
                os0, os1, os2, os3, os4, os5,
                oi0, oi1, oi2, oi3, oi4, oi5, rank_v,
                boxes_v, s_st, l_st, b_st, sem_a, sem_b):
        opt_s_v = (os0, os1, os2, os3, os4, os5)
        opt_i_v = (oi0, oi1, oi2, oi3, oi4, oi5)
        img = lax.axis_index("s") * 2 + lax.axis_index("c")

        bufs = (buf_a, buf_b)
        sems = (sem_a, sem_b)
        copies = [None, None]

        def start_chunk(c):
            copies[c % 2] = pltpu.async_copy(
                logits_hbm.at[img, pl.ds(c * RCH, RCH)], bufs[c % 2],
                sems[c % 2])

        start_chunk(0)
        pltpu.sync_copy(boxes_hbm.at[pl.ds(img * QB, QB)], boxes_v)

        lanes = lax.iota(jnp.int32, L)
        ones = jnp.ones((L,), jnp.int32)
        zeros = jnp.zeros((L,), jnp.int32)
        cvecs = [c0 + lanes for c0 in _C0S]
        last_ok = lanes >= 5

        def zero_hist(i, _):
            for u in range(4):
                hist_v[pl.ds((i * 4 + u) * L, L)] = zeros
            return 0
        lax.fori_loop(0, NBINS // L // 4, zero_hist, 0)

        def p1_row(buf, r, base91, wptrs):
            xs = [buf[r, pl.ds(c0, L)] for c0 in _C0S]
            ss = [_sortable(lax.bitcast_convert_type(x, jnp.int32))
                  for x in xs]
            ms = [s >= THETA_OPT for s in ss]
            ms[5] = jnp.logical_and(ms[5], last_ok)
            new = []
            for si in range(6):
                wp = jnp.minimum(wptrs[si][0], CAP_SI - L)
                plsc.store_compressed(opt_s_v[si].at[pl.ds(wp, L)],
                                      ss[si], mask=ms[si])
                plsc.store_compressed(opt_i_v[si].at[pl.ds(wp, L)],
                                      base91 + cvecs[si], mask=ms[si])
                new.append(wptrs[si] +
                           plsc.all_reduce_population_count(ms[si]))
            return tuple(new)

        wptrs = (zeros,) * 6
        for c in range(NCHUNK):
            if c + 1 < NCHUNK:
                start_chunk(c + 1)
            copies[c % 2].wait()
            buf = bufs[c % 2]

            def p1_rows(r, wv, _buf=buf, _c=c):
                return p1_row(_buf, r, (_c * RCH + r) * C, wv)
            wptrs = lax.fori_loop(0, RCH, p1_rows, wptrs)
        n_sis = [wptrs[si][0] for si in range(6)]
        n_opt = n_sis[0] + n_sis[1] + n_sis[2] + n_sis[3] + n_sis[4] + n_sis[5]
        ok_caps = jnp.logical_and(
            jnp.logical_and(
                jnp.logical_and(n_sis[0] <= CAP_SI - L, n_sis[1] <= CAP_SI - L),
                jnp.logical_and(n_sis[2] <= CAP_SI - L, n_sis[3] <= CAP_SI - L)),
            jnp.logical_and(n_sis[4] <= CAP_SI - L, n_sis[5] <= CAP_SI - L))
        good = jnp.logical_and(n_opt >= K, ok_caps)

        def hist_from_candidates(_):
            for si in range(6):
                nv_si = (n_sis[si] + L - 1) // L
                def body(i, __, _si=si):
                    g = i * L + lanes
                    s = opt_s_v[_si][pl.ds(i * L, L)]
                    bins = jnp.minimum(
                        lax.shift_right_logical(s - THETA_OPT, FINE_SHIFT),
                        NBINS - 1)
                    m = g < n_sis[_si]
                    bins = jnp.where(m, bins, 0)
                    plsc.addupdate_scatter(hist_v, [bins], ones, mask=m)
                    return 0
                lax.fori_loop(0, nv_si, body, 0)
            return 0

        def hist_from_stream(_):
            for c in range(NCHUNK):
                pltpu.sync_copy(
                    logits_hbm.at[img, pl.ds(c * RCH, RCH)], buf_a)
                def fh_row(r, __):
                    for si, c0 in enumerate(_C0S):
                        x = buf_a[r, pl.ds(c0, L)]
                        s = _sortable(lax.bitcast_convert_type(x, jnp.int32))
                        bins = (s >> BIN_SHIFT) + HALF
                        if si == 5:
                            bins = jnp.where(last_ok, bins, 0)
                            plsc.addupdate_scatter(hist_v, [bins], ones,
                                                   mask=last_ok)
                        else:
                            plsc.addupdate_scatter(hist_v, [bins], ones)
                    return 0
                lax.fori_loop(0, RCH, fh_row, 0)
            return 0

        lax.cond(good, hist_from_candidates, hist_from_stream, 0)

        def scan_cond(carry):
            prev, cum, vr = carry
            return jnp.logical_and(cum < K, vr >= 0)

        def scan_chunk(carry):
            prev, cum, vr = carry
            acc = zeros
            for k in range(CH):
                acc = acc + hist_v[pl.ds((vr + k) * L, L)]
            return cum, cum + jnp.sum(acc), vr - CH
        prev, _, vr_exit = lax.while_loop(
            scan_cond, scan_chunk,
            (jnp.int32(0), jnp.int32(0), jnp.int32(NBINS // L - CH)))
        cbase = vr_exit + CH

        def fine_scan(j, carry):
            cum, bstar = carry
            vr = cbase + CH - 1 - j
            v = hist_v[pl.ds(vr * L, L)]
            sfx = jnp.cumsum(lax.rev(v, (0,)))
            tot = jnp.sum(v)
            p = jnp.sum((cum + sfx < K).astype(jnp.int32))
            newcum = cum + tot
            crossed = jnp.logical_and(cum < K, newcum >= K)
            bstar = jnp.where(crossed, vr * L + (L - 1) - p, bstar)
            return newcum, bstar
        _, bstar = lax.fori_loop(0, CH, fine_scan, (prev, jnp.int32(0)))
        theta = jnp.where(
            good, THETA_OPT + lax.shift_left(bstar, FINE_SHIFT),
            lax.shift_left(bstar - HALF, BIN_SHIFT))

        def mini_compact(_):
            wv = zeros
            for si in range(6):
                nv_si = (n_sis[si] + L - 1) // L
                def body(i, w, _si=si):
                    g = i * L + lanes
                    s = opt_s_v[_si][pl.ds(i * L, L)]
                    ci = opt_i_v[_si][pl.ds(i * L, L)]
                    m = jnp.logical_and(s >= theta, g < n_sis[_si])
                    wp = w[0]
                    plsc.store_compressed(cand_s_v.at[pl.ds(wp, L)], s, mask=m)
                    plsc.store_compressed(cand_i_v.at[pl.ds(wp, L)], ci, mask=m)
                    return w + plsc.all_reduce_population_count(m)
                wv = lax.fori_loop(0, nv_si, body, wv)
            return wv[0]

        def full_compact(_):
            wv = zeros
            for c in range(NCHUNK):
                pltpu.sync_copy(
                    logits_hbm.at[img, pl.ds(c * RCH, RCH)], buf_a)

                def fc_row(r, w, _c=c):
                    base91 = (_c * RCH + r) * C
                    for si, c0 in enumerate(_C0S):
                        x = buf_a[r, pl.ds(c0, L)]
                        s = _sortable(lax.bitcast_convert_type(x, jnp.int32))
                        m = s >= theta
                        if si == 5:
                            m = jnp.logical_and(m, last_ok)
                        wp = jnp.minimum(w[0], NCAND - L)
                        plsc.store_compressed(
                            cand_s_v.at[pl.ds(wp, L)], s, mask=m)
                        plsc.store_compressed(
                            cand_i_v.at[pl.ds(wp, L)], base91 + cvecs[si],
                            mask=m)
                        w = w + plsc.all_reduce_population_count(m)
                    return w
                wv = lax.fori_loop(0, RCH, fc_row, wv)
            return wv[0]

        n = lax.cond(good, mini_compact, full_compact, 0)
        n = jnp.minimum(n, NCAND)
        cand_s_v[pl.ds(n, L)] = jnp.full((L,), INT_MIN, jnp.int32)
        cand_i_v[pl.ds(n, L)] = jnp.full((L,), INT_MAX, jnp.int32)
        nv = (n + L - 1) // L

        def rank_outer(iv, _):
            sv = cand_s_v[pl.ds(iv * L, L)]
            ivv = cand_i_v[pl.ds(iv * L, L)]
            def rank_inner(jv, racc):
                sjv = cand_s_v[pl.ds(jv * L, L)]
                ijv = cand_i_v[pl.ds(jv * L, L)]
                for k in range(L):
                    sj = sjv[k]
                    ij = ijv[k]
                    beats = jnp.logical_or(
                        sj > sv, jnp.logical_and(sj == sv, ij < ivv))
                    racc = racc + beats.astype(jnp.int32)
                return racc
            rank_v[pl.ds(iv * L, L)] = lax.fori_loop(0, nv, rank_inner, zeros)
            return 0
        lax.fori_loop(0, nv, rank_outer, 0)

        def emit(iv, _):
            base = iv * L
            r = rank_v[pl.ds(base, L)]
            s = cand_s_v[pl.ds(base, L)]
            ci = cand_i_v[pl.ds(base, L)]
            m = r < K
            rr = jnp.where(m, r, 0)
            x = lax.bitcast_convert_type(_sortable(s), jnp.float32)
            score = 1.0 / (1.0 + jnp.exp(-x))
            q = ((ci.astype(jnp.float32) + 0.5) * inv_c).astype(jnp.int32)
            q = jnp.where(m, q, 0)
            label = ci - q * C
            qb = q * 4
            cx = plsc.load_gather(boxes_v, [qb])
            cy = plsc.load_gather(boxes_v, [qb + 1])
            w = jnp.maximum(plsc.load_gather(boxes_v, [qb + 2]), 0.0)
            h = jnp.maximum(plsc.load_gather(boxes_v, [qb + 3]), 0.0)
            plsc.store_scatter(s_st, [rr], score, mask=m)
            plsc.store_scatter(l_st, [rr], label, mask=m)
            rb = rr * 4
            plsc.store_scatter(b_st, [rb], cx - 0.5 * w, mask=m)
            plsc.store_scatter(b_st, [rb + 1], cy - 0.5 * h, mask=m)
            plsc.store_scatter(b_st, [rb + 2], cx + 0.5 * w, mask=m)
            plsc.store_scatter(b_st, [rb + 3], cy + 0.5 * h, mask=m)
            return 0
        lax.fori_loop(0, nv, emit, 0)

        pltpu.sync_copy(s_st, out_s_hbm.at[pl.ds(img * KSP, KSP)])
        pltpu.sync_copy(l_st, out_l_hbm.at[pl.ds(img * KSP, KSP)])
        pltpu.sync_copy(b_st, out_b_hbm.at[pl.ds(img * KSP * 4, KSP * 4)])

    return sc_call


def kernel(pred_logits, pred_boxes, target_sizes):
    B, Q, C = pred_logits.shape
    boxes1d = pred_boxes.reshape(B * Q * 4)
    s_pad, l_pad, b_pad = _make_sc_call(B, Q, C)(pred_logits, boxes1d)
    scores = s_pad.reshape(B, KSP)[:, :K]
    labels = l_pad.reshape(B, KSP)[:, :K]
    ts = target_sizes.astype(jnp.float32)
    scale_fct = jnp.stack([ts[:, 1], ts[:, 0], ts[:, 1], ts[:, 0]],
                          axis=-1)[:, None, :]
    boxes = b_pad.reshape(B, KSP, 4)[:, :K, :] * scale_fct
    return scores, labels, boxes
